# baseline (speedup 1.0000x reference)
import functools

import jax
import jax.numpy as jnp
from jax import lax
from jax.experimental import pallas as pl
from jax.experimental.pallas import tpu as pltpu
from jax.experimental.pallas import tpu_sc as plsc

_V = 40
_D = 128
_CHUNK = 128
_NW = 32
_STEPS = 200
_Q = _STEPS // 2
_BPW = _STEPS * _CHUNK
_SUB = _CHUNK // 16


def _make_sc_gather():
    mesh = plsc.VectorSubcoreMesh(core_axis_name="c", subcore_axis_name="s")

    @functools.partial(
        pl.kernel,
        mesh=mesh,
        compiler_params=pltpu.CompilerParams(needs_layout_passes=False),
        out_type=jax.ShapeDtypeStruct((_NW, _BPW * _D), jnp.float32),
        scratch_types=[
            pltpu.VMEM((_V * _D,), jnp.float32),
            pltpu.VMEM((_BPW,), jnp.int32),
            pltpu.VMEM((_CHUNK * _D,), jnp.float32),
            pltpu.VMEM((_CHUNK * _D,), jnp.float32),
            pltpu.SemaphoreType.DMA,
            pltpu.SemaphoreType.DMA,
        ],
    )
    def sc_gather(table_hbm, idx_hbm, out_hbm, tbl_v, idx_v, rows0, rows1,
                  w_sem0, w_sem1):
        wid = lax.axis_index("s") * 2 + lax.axis_index("c")
        pltpu.sync_copy(table_hbm, tbl_v)
        pltpu.sync_copy(idx_hbm.at[wid], idx_v)

        sa_base = lax.iota(jnp.int32, 16) * _D

        def compute_chunk(j, buf):
            def sub(s, carry):
                rvec = idx_v[pl.ds(j * _CHUNK + s * 16, 16)]
                la = rvec * _D
                sa = sa_base + s * (16 * _D)
                for _ in range(_D):
                    v = plsc.load_gather(tbl_v, [la])
                    plsc.store_scatter(buf, [sa], v)
                    la = la + 1
                    sa = sa + 1
                return carry

            lax.fori_loop(0, _SUB, sub, None)

        def write_chunk(j, buf, w_sem):
            pltpu.async_copy(
                buf, out_hbm.at[wid, pl.ds(j * _CHUNK * _D, _CHUNK * _D)],
                w_sem)

        def drain_write(buf, w_sem):
            pltpu.make_async_copy(
                buf, out_hbm.at[wid, pl.ds(0, _CHUNK * _D)], w_sem).wait()

        def qstep(q, carry):
            @pl.when(q > 0)
            def _():
                drain_write(rows0, w_sem0)

            compute_chunk(2 * q, rows0)
            write_chunk(2 * q, rows0, w_sem0)

            @pl.when(q > 0)
            def _():
                drain_write(rows1, w_sem1)

            compute_chunk(2 * q + 1, rows1)
            write_chunk(2 * q + 1, rows1, w_sem1)
            return carry

        lax.fori_loop(0, _Q, qstep, None)
        drain_write(rows0, w_sem0)
        drain_write(rows1, w_sem1)

    return sc_gather


_sc_gather = _make_sc_gather()


def kernel(indices, embeddings):
    b, l = indices.shape
    idx = indices.reshape(_NW, _BPW)
    out = _sc_gather(embeddings.reshape(-1), idx)
    return out.reshape(b, l, _D)


# speedup vs baseline: 1.2811x; 1.2811x over previous
import functools

import jax
import jax.numpy as jnp
from jax import lax
from jax.experimental import pallas as pl
from jax.experimental.pallas import tpu as pltpu
from jax.experimental.pallas import tpu_sc as plsc

_V = 40
_D = 128
_CHUNK = 128
_NW = 32
_STEPS = 200
_Q = _STEPS // 2
_BPW = _STEPS * _CHUNK
_SUB = _CHUNK // 16


def _make_sc_gather():
    mesh = plsc.VectorSubcoreMesh(core_axis_name="c", subcore_axis_name="s")

    @functools.partial(
        pl.kernel,
        mesh=mesh,
        compiler_params=pltpu.CompilerParams(needs_layout_passes=False),
        out_type=jax.ShapeDtypeStruct((_NW, _BPW * _D), jnp.float32),
        scratch_types=[
            pltpu.VMEM((_V * _D,), jnp.float32),
            pltpu.VMEM((_BPW,), jnp.int32),
            pltpu.VMEM((_CHUNK * _D,), jnp.float32),
            pltpu.VMEM((_CHUNK * _D,), jnp.float32),
            pltpu.SemaphoreType.DMA,
            pltpu.SemaphoreType.DMA,
        ],
    )
    def sc_gather(table_hbm, idx_hbm, out_hbm, tbl_v, idx_v, rows0, rows1,
                  w_sem0, w_sem1):
        wid = lax.axis_index("s") * 2 + lax.axis_index("c")
        pltpu.sync_copy(table_hbm, tbl_v)
        pltpu.sync_copy(idx_hbm.at[wid], idx_v)

        def compute_chunk(j, buf):
            def sub(s, carry):
                rvec = idx_v[pl.ds(j * _CHUNK + s * 16, 16)]
                obase = s * (16 * _D)
                for i in range(16):
                    rbase = rvec[i] * _D
                    for c in range(_D // 16):
                        buf[pl.ds(obase + i * _D + c * 16, 16)] = (
                            tbl_v[pl.ds(rbase + c * 16, 16)])
                return carry

            lax.fori_loop(0, _SUB, sub, None)

        def write_chunk(j, buf, w_sem):
            pltpu.async_copy(
                buf, out_hbm.at[wid, pl.ds(j * _CHUNK * _D, _CHUNK * _D)],
                w_sem)

        def drain_write(buf, w_sem):
            pltpu.make_async_copy(
                buf, out_hbm.at[wid, pl.ds(0, _CHUNK * _D)], w_sem).wait()

        def qstep(q, carry):
            @pl.when(q > 0)
            def _():
                drain_write(rows0, w_sem0)

            compute_chunk(2 * q, rows0)
            write_chunk(2 * q, rows0, w_sem0)

            @pl.when(q > 0)
            def _():
                drain_write(rows1, w_sem1)

            compute_chunk(2 * q + 1, rows1)
            write_chunk(2 * q + 1, rows1, w_sem1)
            return carry

        lax.fori_loop(0, _Q, qstep, None)
        drain_write(rows0, w_sem0)
        drain_write(rows1, w_sem1)

    return sc_gather


_sc_gather = _make_sc_gather()


def kernel(indices, embeddings):
    b, l = indices.shape
    idx = indices.reshape(_NW, _BPW)
    out = _sc_gather(embeddings.reshape(-1), idx)
    return out.reshape(b, l, _D)


# speedup vs baseline: 71.5026x; 55.8135x over previous
import functools

import jax
import jax.numpy as jnp
from jax import lax
from jax.experimental import pallas as pl
from jax.experimental.pallas import tpu as pltpu
from jax.experimental.pallas import tpu_sc as plsc

_V = 40
_D = 128
_CHUNK = 128
_K = 2
_GROUP = _K * _CHUNK
_NW = 32
_STEPS = 200
_PHASES = _STEPS // _K
_Q = _PHASES // 2
_BPW = _STEPS * _CHUNK


def _make_sc_gather():
    mesh = plsc.VectorSubcoreMesh(core_axis_name="c", subcore_axis_name="s")

    @functools.partial(
        pl.kernel,
        mesh=mesh,
        out_type=jax.ShapeDtypeStruct((_NW, _BPW, _D), jnp.float32),
        scratch_types=[
            pltpu.VMEM_SHARED((_V, _D), jnp.float32),
            pltpu.VMEM((_STEPS, _CHUNK), jnp.int32),
            pltpu.VMEM((2, _GROUP, _D), jnp.float32),
            pltpu.SemaphoreType.DMA,
            pltpu.SemaphoreType.DMA,
            pltpu.SemaphoreType.DMA,
        ],
    )
    def sc_gather(table_hbm, idx_hbm, out_hbm, tbl_s, idx_v, rows_v, g_sem,
                  w_sem0, w_sem1):
        sid = lax.axis_index("s")
        wid = sid * 2 + lax.axis_index("c")

        @pl.when(sid == 0)
        def _():
            pltpu.sync_copy(table_hbm, tbl_s)

        pltpu.sync_copy(idx_hbm.at[wid], idx_v)
        plsc.subcore_barrier()

        def issue_gathers(phase, grp):
            for b in range(_K):
                pltpu.async_copy(
                    tbl_s.at[idx_v.at[phase * _K + b]],
                    rows_v.at[grp, pl.ds(b * _CHUNK, _CHUNK)],
                    g_sem,
                )

        def drain_gathers(grp):
            for b in range(_K):
                pltpu.make_async_copy(
                    tbl_s.at[idx_v.at[0]],
                    rows_v.at[grp, pl.ds(b * _CHUNK, _CHUNK)],
                    g_sem,
                ).wait()

        def write_group(phase, grp, w_sem):
            pltpu.async_copy(
                rows_v.at[grp],
                out_hbm.at[wid, pl.ds(phase * _GROUP, _GROUP)],
                w_sem,
            )

        def drain_write(grp, w_sem):
            pltpu.make_async_copy(
                rows_v.at[grp],
                out_hbm.at[wid, pl.ds(0, _GROUP)],
                w_sem,
            ).wait()

        issue_gathers(0, 0)

        def qstep(q, carry):
            p0 = 2 * q
            drain_gathers(0)
            write_group(p0, 0, w_sem0)

            @pl.when(q > 0)
            def _():
                drain_write(1, w_sem1)

            issue_gathers(p0 + 1, 1)

            drain_gathers(1)
            write_group(p0 + 1, 1, w_sem1)
            drain_write(0, w_sem0)

            @pl.when(q < _Q - 1)
            def _():
                issue_gathers(p0 + 2, 0)

            return carry

        lax.fori_loop(0, _Q, qstep, None)
        drain_write(1, w_sem1)

    return sc_gather


_sc_gather = _make_sc_gather()


def kernel(indices, embeddings):
    b, l = indices.shape
    idx = indices.reshape(_NW, _STEPS, _CHUNK)
    out = _sc_gather(embeddings, idx)
    return out.reshape(b, l, _D)


# speedup vs baseline: 74.4107x; 1.0407x over previous
import functools

import jax
import jax.numpy as jnp
from jax import lax
from jax.experimental import pallas as pl
from jax.experimental.pallas import tpu as pltpu
from jax.experimental.pallas import tpu_sc as plsc

_V = 40
_D = 128
_CHUNK = 128
_NB = 5
_NW = 32
_STEPS = 200
_Q = _STEPS // _NB
_BPW = _STEPS * _CHUNK


def _make_sc_gather():
    mesh = plsc.VectorSubcoreMesh(core_axis_name="c", subcore_axis_name="s")

    @functools.partial(
        pl.kernel,
        mesh=mesh,
        out_type=jax.ShapeDtypeStruct((_NW, _BPW, _D), jnp.float32),
        scratch_types=[
            pltpu.VMEM_SHARED((_V, _D), jnp.float32),
            pltpu.VMEM((_STEPS, _CHUNK), jnp.int32),
            pltpu.VMEM((_NB, _CHUNK, _D), jnp.float32),
            pltpu.SemaphoreType.DMA,
        ] + [pltpu.SemaphoreType.DMA] * _NB,
    )
    def sc_gather(table_hbm, idx_hbm, out_hbm, tbl_s, idx_v, rows_v, g_sem,
                  *w_sems):
        sid = lax.axis_index("s")
        wid = sid * 2 + lax.axis_index("c")

        @pl.when(sid == 0)
        def _():
            pltpu.sync_copy(table_hbm, tbl_s)

        pltpu.sync_copy(idx_hbm.at[wid], idx_v)
        plsc.subcore_barrier()

        def issue_gather(j, u):
            pltpu.async_copy(tbl_s.at[idx_v.at[j]], rows_v.at[u], g_sem)

        def drain_gather(u):
            pltpu.make_async_copy(
                tbl_s.at[idx_v.at[0]], rows_v.at[u], g_sem).wait()

        def issue_write(j, u):
            pltpu.async_copy(
                rows_v.at[u], out_hbm.at[wid, pl.ds(j * _CHUNK, _CHUNK)],
                w_sems[u])

        def drain_write(u):
            pltpu.make_async_copy(
                rows_v.at[u], out_hbm.at[wid, pl.ds(0, _CHUNK)],
                w_sems[u]).wait()

        issue_gather(0, 0)
        issue_gather(1, 1)

        def qstep(q, carry):
            for u in range(_NB):
                j = _NB * q + u
                drain_gather(u)
                issue_write(j, u)
                un = (u + 2) % _NB

                @pl.when(j >= 3)
                def _():
                    drain_write(un)

                @pl.when(j < _STEPS - 2)
                def _():
                    issue_gather(j + 2, un)

            return carry

        lax.fori_loop(0, _Q, qstep, None)
        for u in ((_STEPS - 3) % _NB, (_STEPS - 2) % _NB, (_STEPS - 1) % _NB):
            drain_write(u)

    return sc_gather


_sc_gather = _make_sc_gather()


def kernel(indices, embeddings):
    b, l = indices.shape
    idx = indices.reshape(_NW, _STEPS, _CHUNK)
    out = _sc_gather(embeddings, idx)
    return out.reshape(b, l, _D)
